# baseline (speedup 1.0000x reference)
import functools

import jax
import jax.numpy as jnp
from jax import lax
from jax.experimental import pallas as pl
from jax.experimental.pallas import tpu as pltpu
from jax.experimental.pallas import tpu_sc as plsc

_B, _T, _D, _CD, _K = 16, 1024, 768, 256, 8192
_N = _B * _T

_R1 = 256
_G1 = _N // _R1


def _front_body(feat, w1, b1, w2, b2, g, beta, cbt, cw, idx_out, enc_out,
                e2_scr):
    @pl.when(pl.program_id(0) == 0)
    def _():
        c = cbt[...]
        e2_scr[...] = jnp.sum(c * c, axis=0, keepdims=True)

    x = feat[...]
    h = jnp.maximum(jnp.dot(x, w1[...], preferred_element_type=jnp.float32)
                    + b1[...], 0.0)
    h = jnp.dot(h, w2[...], preferred_element_type=jnp.float32) + b2[...]
    mu = jnp.mean(h, axis=1, keepdims=True)
    var = jnp.mean((h - mu) ** 2, axis=1, keepdims=True)
    flat = (h - mu) / jnp.sqrt(var + 1e-5) * g[...] + beta[...]

    x2 = jnp.sum(flat * flat, axis=1, keepdims=True)
    m2 = jnp.dot(flat * (-2.0), cbt[...], preferred_element_type=jnp.float32)
    d = (x2 + e2_scr[...]) + m2
    dmin = jnp.min(d, axis=1, keepdims=True)
    hit = d == dmin
    enc = hit.astype(jnp.float32)
    enc_out[...] = enc
    cs = jnp.dot(enc, cw[...], preferred_element_type=jnp.float32)
    cnt = cs[:, 0:1]
    idx_f = cs[:, 1:2] * 64.0 + cs[:, 2:3]
    idx_out[0, :, :] = idx_f.astype(jnp.int32)

    @pl.when(jnp.any(cnt != 1.0))
    def _():
        iota = lax.broadcasted_iota(jnp.int32, (_R1, _K), 1)
        idxs = jnp.min(jnp.where(hit, iota, _K), axis=1, keepdims=True)
        idx_out[0, :, :] = idxs
        enc_out[...] = (iota == idxs).astype(jnp.float32)


_front = pl.pallas_call(
    _front_body,
    grid=(_G1,),
    in_specs=[
        pl.BlockSpec((_R1, _D), lambda i: (i, 0)),
        pl.BlockSpec((_D, _D), lambda i: (0, 0)),
        pl.BlockSpec((1, _D), lambda i: (0, 0)),
        pl.BlockSpec((_D, _CD), lambda i: (0, 0)),
        pl.BlockSpec((1, _CD), lambda i: (0, 0)),
        pl.BlockSpec((1, _CD), lambda i: (0, 0)),
        pl.BlockSpec((1, _CD), lambda i: (0, 0)),
        pl.BlockSpec((_CD, _K), lambda i: (0, 0)),
        pl.BlockSpec((_K, 4), lambda i: (0, 0)),
    ],
    out_specs=[
        pl.BlockSpec((1, _R1, 1), lambda i: (i, 0, 0)),
        pl.BlockSpec((_R1, _K), lambda i: (i, 0)),
    ],
    out_shape=[
        jax.ShapeDtypeStruct((_G1, _R1, 1), jnp.int32),
        jax.ShapeDtypeStruct((_N, _K), jnp.float32),
    ],
    scratch_shapes=[pltpu.VMEM((1, _K), jnp.float32)],
    compiler_params=pltpu.CompilerParams(dimension_semantics=("arbitrary",)),
)

_NC, _NS = 2, 16
_NW = _NC * _NS
_RPW = _N // _NW
_CH = 128
_NCH = _RPW // _CH


def _gather_body(cb_hbm, idx_hbm, out_hbm, idx_v, rows_v, sem):
    wid = lax.axis_index("s") * _NC + lax.axis_index("c")
    for ch in range(_NCH):
        base = wid * _RPW + ch * _CH
        pltpu.sync_copy(idx_hbm.at[pl.ds(base, _CH)], idx_v)
        pltpu.async_copy(cb_hbm.at[idx_v], rows_v, sem).wait()
        pltpu.sync_copy(rows_v, out_hbm.at[pl.ds(base, _CH)])


@functools.cache
def _build_gather():
    return functools.partial(
        pl.kernel,
        out_type=jax.ShapeDtypeStruct((_N, _CD), jnp.float32),
        mesh=plsc.VectorSubcoreMesh(core_axis_name="c", subcore_axis_name="s"),
        scratch_types=[
            pltpu.VMEM((_CH,), jnp.int32),
            pltpu.VMEM((_CH, _CD), jnp.float32),
            pltpu.SemaphoreType.DMA,
        ],
    )(_gather_body)


def _gather(cb, idx):
    return _build_gather()(cb, idx)

_R3 = 1024
_G3 = _N // _R3


def _back_body(qr, wo1, bo1, wo2, bo2, g, beta, out):
    h = jnp.maximum(jnp.dot(qr[...], wo1[...], preferred_element_type=jnp.float32)
                    + bo1[...], 0.0)
    h = jnp.dot(h, wo2[...], preferred_element_type=jnp.float32) + bo2[...]
    mu = jnp.mean(h, axis=1, keepdims=True)
    var = jnp.mean((h - mu) ** 2, axis=1, keepdims=True)
    out[...] = (h - mu) / jnp.sqrt(var + 1e-5) * g[...] + beta[...]


_back = pl.pallas_call(
    _back_body,
    grid=(_G3,),
    in_specs=[
        pl.BlockSpec((_R3, _CD), lambda i: (i, 0)),
        pl.BlockSpec((_CD, _D), lambda i: (0, 0)),
        pl.BlockSpec((1, _D), lambda i: (0, 0)),
        pl.BlockSpec((_D, _D), lambda i: (0, 0)),
        pl.BlockSpec((1, _D), lambda i: (0, 0)),
        pl.BlockSpec((1, _D), lambda i: (0, 0)),
        pl.BlockSpec((1, _D), lambda i: (0, 0)),
    ],
    out_specs=pl.BlockSpec((_R3, _D), lambda i: (i, 0)),
    out_shape=jax.ShapeDtypeStruct((_N, _D), jnp.float32),
    compiler_params=pltpu.CompilerParams(dimension_semantics=("arbitrary",)),
)


def kernel(features, W_in1, b_in1, W_in2, b_in2, g_nin, beta_nin, codebook,
           W_out1, b_out1, W_out2, b_out2, g_nout, beta_nout):
    feat = features.reshape(_N, _D)
    cbt = codebook.T
    j = jnp.arange(_K, dtype=jnp.float32)
    cw = jnp.stack([jnp.ones(_K, jnp.float32), jnp.floor(j / 64.0),
                    jnp.mod(j, 64.0), jnp.zeros(_K, jnp.float32)], axis=1)
    idx3, enc = _front(feat, W_in1, b_in1.reshape(1, -1), W_in2,
                       b_in2.reshape(1, -1), g_nin.reshape(1, -1),
                       beta_nin.reshape(1, -1), cbt, cw)
    idx_flat = idx3.reshape(_N)
    qr = _gather(codebook, idx_flat)
    q = _back(qr, W_out1, b_out1.reshape(1, -1), W_out2,
              b_out2.reshape(1, -1), g_nout.reshape(1, -1),
              beta_nout.reshape(1, -1))
    return q.reshape(_B, _T, _D), idx_flat.reshape(-1, 1), enc


# speedup vs baseline: 1.4266x; 1.4266x over previous
import functools

import jax
import jax.numpy as jnp
from jax import lax
from jax.experimental import pallas as pl
from jax.experimental.pallas import tpu as pltpu
from jax.experimental.pallas import tpu_sc as plsc

_B, _T, _D, _CD, _K = 16, 1024, 768, 256, 8192
_N = _B * _T

_R1 = 256
_G1 = _N // _R1


def _front_body(feat, w1, b1, w2, b2, g, beta, cbt, idx_out, enc_out,
                e2_scr):
    @pl.when(pl.program_id(0) == 0)
    def _():
        c = cbt[...]
        e2_scr[...] = jnp.sum(c * c, axis=0, keepdims=True)

    x = feat[...]
    h = jnp.maximum(jnp.dot(x, w1[...], preferred_element_type=jnp.float32)
                    + b1[...], 0.0)
    h = jnp.dot(h, w2[...], preferred_element_type=jnp.float32) + b2[...]
    mu = jnp.mean(h, axis=1, keepdims=True)
    var = jnp.mean((h - mu) ** 2, axis=1, keepdims=True)
    flat = (h - mu) / jnp.sqrt(var + 1e-5) * g[...] + beta[...]

    x2 = jnp.sum(flat * flat, axis=1, keepdims=True)
    m2 = jnp.dot(flat * (-2.0), cbt[...], preferred_element_type=jnp.float32)
    d = (x2 + e2_scr[...]) + m2
    idxs = jnp.argmin(d, axis=1)[:, None]
    idx_out[0, :, :] = idxs
    iota = lax.broadcasted_iota(jnp.int32, (_R1, _K), 1)
    enc_out[...] = (iota == idxs).astype(jnp.float32)


_front = pl.pallas_call(
    _front_body,
    grid=(_G1,),
    in_specs=[
        pl.BlockSpec((_R1, _D), lambda i: (i, 0)),
        pl.BlockSpec((_D, _D), lambda i: (0, 0)),
        pl.BlockSpec((1, _D), lambda i: (0, 0)),
        pl.BlockSpec((_D, _CD), lambda i: (0, 0)),
        pl.BlockSpec((1, _CD), lambda i: (0, 0)),
        pl.BlockSpec((1, _CD), lambda i: (0, 0)),
        pl.BlockSpec((1, _CD), lambda i: (0, 0)),
        pl.BlockSpec((_CD, _K), lambda i: (0, 0)),
    ],
    out_specs=[
        pl.BlockSpec((1, _R1, 1), lambda i: (i, 0, 0)),
        pl.BlockSpec((_R1, _K), lambda i: (i, 0)),
    ],
    out_shape=[
        jax.ShapeDtypeStruct((_G1, _R1, 1), jnp.int32),
        jax.ShapeDtypeStruct((_N, _K), jnp.float32),
    ],
    scratch_shapes=[pltpu.VMEM((1, _K), jnp.float32)],
    compiler_params=pltpu.CompilerParams(dimension_semantics=("arbitrary",)),
)

_NC, _NS = 2, 16
_NW = _NC * _NS
_RPW = _N // _NW
_CH = 128
_NCH = _RPW // _CH


def _gather_body(cb_hbm, idx_hbm, out_hbm, idx_v, rows_v, sem):
    wid = lax.axis_index("s") * _NC + lax.axis_index("c")
    for ch in range(_NCH):
        base = wid * _RPW + ch * _CH
        pltpu.sync_copy(idx_hbm.at[pl.ds(base, _CH)], idx_v)
        pltpu.async_copy(cb_hbm.at[idx_v], rows_v, sem).wait()
        pltpu.sync_copy(rows_v, out_hbm.at[pl.ds(base, _CH)])


@functools.cache
def _build_gather():
    return functools.partial(
        pl.kernel,
        out_type=jax.ShapeDtypeStruct((_N, _CD), jnp.float32),
        mesh=plsc.VectorSubcoreMesh(core_axis_name="c", subcore_axis_name="s"),
        scratch_types=[
            pltpu.VMEM((_CH,), jnp.int32),
            pltpu.VMEM((_CH, _CD), jnp.float32),
            pltpu.SemaphoreType.DMA,
        ],
    )(_gather_body)


def _gather(cb, idx):
    return _build_gather()(cb, idx)

_R3 = 1024
_G3 = _N // _R3


def _back_body(qr, wo1, bo1, wo2, bo2, g, beta, out):
    h = jnp.maximum(jnp.dot(qr[...], wo1[...], preferred_element_type=jnp.float32)
                    + bo1[...], 0.0)
    h = jnp.dot(h, wo2[...], preferred_element_type=jnp.float32) + bo2[...]
    mu = jnp.mean(h, axis=1, keepdims=True)
    var = jnp.mean((h - mu) ** 2, axis=1, keepdims=True)
    out[...] = (h - mu) / jnp.sqrt(var + 1e-5) * g[...] + beta[...]


_back = pl.pallas_call(
    _back_body,
    grid=(_G3,),
    in_specs=[
        pl.BlockSpec((_R3, _CD), lambda i: (i, 0)),
        pl.BlockSpec((_CD, _D), lambda i: (0, 0)),
        pl.BlockSpec((1, _D), lambda i: (0, 0)),
        pl.BlockSpec((_D, _D), lambda i: (0, 0)),
        pl.BlockSpec((1, _D), lambda i: (0, 0)),
        pl.BlockSpec((1, _D), lambda i: (0, 0)),
        pl.BlockSpec((1, _D), lambda i: (0, 0)),
    ],
    out_specs=pl.BlockSpec((_R3, _D), lambda i: (i, 0)),
    out_shape=jax.ShapeDtypeStruct((_N, _D), jnp.float32),
    compiler_params=pltpu.CompilerParams(dimension_semantics=("arbitrary",)),
)


def kernel(features, W_in1, b_in1, W_in2, b_in2, g_nin, beta_nin, codebook,
           W_out1, b_out1, W_out2, b_out2, g_nout, beta_nout):
    feat = features.reshape(_N, _D)
    cbt = codebook.T
    idx3, enc = _front(feat, W_in1, b_in1.reshape(1, -1), W_in2,
                       b_in2.reshape(1, -1), g_nin.reshape(1, -1),
                       beta_nin.reshape(1, -1), cbt)
    idx_flat = idx3.reshape(_N)
    qr = _gather(codebook, idx_flat)
    q = _back(qr, W_out1, b_out1.reshape(1, -1), W_out2,
              b_out2.reshape(1, -1), g_nout.reshape(1, -1),
              beta_nout.reshape(1, -1))
    return q.reshape(_B, _T, _D), idx_flat.reshape(-1, 1), enc


# speedup vs baseline: 1.4299x; 1.0023x over previous
import functools

import jax
import jax.numpy as jnp
from jax import lax
from jax.experimental import pallas as pl
from jax.experimental.pallas import tpu as pltpu
from jax.experimental.pallas import tpu_sc as plsc

_B, _T, _D, _CD, _K = 16, 1024, 768, 256, 8192
_N = _B * _T

_R1 = 256
_G1 = _N // _R1


def _front_body(feat, w1, b1, w2, b2, g, beta, cbt, idx_out, enc_out,
                e2_scr):
    @pl.when(pl.program_id(0) == 0)
    def _():
        c = cbt[...]
        e2_scr[...] = jnp.sum(c * c, axis=0, keepdims=True)

    x = feat[...]
    h = jnp.maximum(jnp.dot(x, w1[...], preferred_element_type=jnp.float32)
                    + b1[...], 0.0)
    h = jnp.dot(h, w2[...], preferred_element_type=jnp.float32) + b2[...]
    mu = jnp.mean(h, axis=1, keepdims=True)
    var = jnp.mean((h - mu) ** 2, axis=1, keepdims=True)
    flat = (h - mu) / jnp.sqrt(var + 1e-5) * g[...] + beta[...]

    x2 = jnp.sum(flat * flat, axis=1, keepdims=True)
    m2 = jnp.dot(flat * (-2.0), cbt[...], preferred_element_type=jnp.float32)
    d = (x2 + e2_scr[...]) + m2
    idxs = jnp.argmin(d, axis=1)[:, None]
    idx_out[0, :, :] = idxs
    iota = lax.broadcasted_iota(jnp.int32, (_R1, _K), 1)
    enc_out[...] = (iota == idxs).astype(jnp.float32)


_front = pl.pallas_call(
    _front_body,
    grid=(_G1,),
    in_specs=[
        pl.BlockSpec((_R1, _D), lambda i: (i, 0)),
        pl.BlockSpec((_D, _D), lambda i: (0, 0)),
        pl.BlockSpec((1, _D), lambda i: (0, 0)),
        pl.BlockSpec((_D, _CD), lambda i: (0, 0)),
        pl.BlockSpec((1, _CD), lambda i: (0, 0)),
        pl.BlockSpec((1, _CD), lambda i: (0, 0)),
        pl.BlockSpec((1, _CD), lambda i: (0, 0)),
        pl.BlockSpec((_CD, _K), lambda i: (0, 0)),
    ],
    out_specs=[
        pl.BlockSpec((1, _R1, 1), lambda i: (i, 0, 0)),
        pl.BlockSpec((_R1, _K), lambda i: (i, 0)),
    ],
    out_shape=[
        jax.ShapeDtypeStruct((_G1, _R1, 1), jnp.int32),
        jax.ShapeDtypeStruct((_N, _K), jnp.float32),
    ],
    scratch_shapes=[pltpu.VMEM((1, _K), jnp.float32)],
    compiler_params=pltpu.CompilerParams(dimension_semantics=("arbitrary",)),
)

_NC, _NS = 2, 16
_NW = _NC * _NS
_RPW = _N // _NW
_CH = 128
_NCH = _RPW // _CH


def _gather_body(cb_hbm, idx_hbm, out_hbm, idx_v, rows_v, gsem, ssem):
    wid = lax.axis_index("s") * _NC + lax.axis_index("c")
    base = wid * _RPW
    pltpu.sync_copy(idx_hbm.at[pl.ds(base, _RPW)], idx_v)

    def g(ch, buf):
        return pltpu.async_copy(
            cb_hbm.at[idx_v.at[pl.ds(ch * _CH, _CH)]], rows_v.at[buf], gsem)

    def s(ch, buf):
        return pltpu.async_copy(
            rows_v.at[buf], out_hbm.at[pl.ds(base + ch * _CH, _CH)], ssem)

    g0 = g(0, 0)
    g1 = g(1, 1)
    g0.wait()
    s0 = s(0, 0)
    g2 = g(2, 2)
    g1.wait()
    s1 = s(1, 1)
    s0.wait()
    g3 = g(3, 0)
    g2.wait()
    s2 = s(2, 2)
    g3.wait()
    s3 = s(3, 0)
    s1.wait()
    s2.wait()
    s3.wait()


@functools.cache
def _build_gather():
    return functools.partial(
        pl.kernel,
        out_type=jax.ShapeDtypeStruct((_N, _CD), jnp.float32),
        mesh=plsc.VectorSubcoreMesh(core_axis_name="c", subcore_axis_name="s"),
        scratch_types=[
            pltpu.VMEM((_RPW,), jnp.int32),
            pltpu.VMEM((3, _CH, _CD), jnp.float32),
            pltpu.SemaphoreType.DMA,
            pltpu.SemaphoreType.DMA,
        ],
    )(_gather_body)


def _gather(cb, idx):
    return _build_gather()(cb, idx)

_R3 = 1024
_G3 = _N // _R3


def _back_body(qr, wo1, bo1, wo2, bo2, g, beta, out):
    h = jnp.maximum(jnp.dot(qr[...], wo1[...], preferred_element_type=jnp.float32)
                    + bo1[...], 0.0)
    h = jnp.dot(h, wo2[...], preferred_element_type=jnp.float32) + bo2[...]
    mu = jnp.mean(h, axis=1, keepdims=True)
    var = jnp.mean((h - mu) ** 2, axis=1, keepdims=True)
    out[...] = (h - mu) / jnp.sqrt(var + 1e-5) * g[...] + beta[...]


_back = pl.pallas_call(
    _back_body,
    grid=(_G3,),
    in_specs=[
        pl.BlockSpec((_R3, _CD), lambda i: (i, 0)),
        pl.BlockSpec((_CD, _D), lambda i: (0, 0)),
        pl.BlockSpec((1, _D), lambda i: (0, 0)),
        pl.BlockSpec((_D, _D), lambda i: (0, 0)),
        pl.BlockSpec((1, _D), lambda i: (0, 0)),
        pl.BlockSpec((1, _D), lambda i: (0, 0)),
        pl.BlockSpec((1, _D), lambda i: (0, 0)),
    ],
    out_specs=pl.BlockSpec((_R3, _D), lambda i: (i, 0)),
    out_shape=jax.ShapeDtypeStruct((_N, _D), jnp.float32),
    compiler_params=pltpu.CompilerParams(dimension_semantics=("arbitrary",)),
)


def kernel(features, W_in1, b_in1, W_in2, b_in2, g_nin, beta_nin, codebook,
           W_out1, b_out1, W_out2, b_out2, g_nout, beta_nout):
    feat = features.reshape(_N, _D)
    cbt = codebook.T
    idx3, enc = _front(feat, W_in1, b_in1.reshape(1, -1), W_in2,
                       b_in2.reshape(1, -1), g_nin.reshape(1, -1),
                       beta_nin.reshape(1, -1), cbt)
    idx_flat = idx3.reshape(_N)
    qr = _gather(codebook, idx_flat)
    q = _back(qr, W_out1, b_out1.reshape(1, -1), W_out2,
              b_out2.reshape(1, -1), g_nout.reshape(1, -1),
              beta_nout.reshape(1, -1))
    return q.reshape(_B, _T, _D), idx_flat.reshape(-1, 1), enc


# speedup vs baseline: 1.5418x; 1.0783x over previous
import functools

import jax
import jax.numpy as jnp
from jax import lax
from jax.experimental import pallas as pl
from jax.experimental.pallas import tpu as pltpu
from jax.experimental.pallas import tpu_sc as plsc

_B, _T, _D, _CD, _K = 16, 1024, 768, 256, 8192
_N = _B * _T

_R1 = 512
_G1 = _N // _R1


def _front_body(feat, w1, b1, w2, b2, g, beta, cbt, idx_out, enc_out,
                e2_scr):
    @pl.when(pl.program_id(0) == 0)
    def _():
        c = cbt[...]
        e2_scr[...] = jnp.sum(c * c, axis=0, keepdims=True)

    x = feat[...]
    h = jnp.maximum(jnp.dot(x, w1[...], preferred_element_type=jnp.float32)
                    + b1[...], 0.0)
    h = jnp.dot(h, w2[...], preferred_element_type=jnp.float32) + b2[...]
    mu = jnp.mean(h, axis=1, keepdims=True)
    var = jnp.mean((h - mu) ** 2, axis=1, keepdims=True)
    flat = (h - mu) / jnp.sqrt(var + 1e-5) * g[...] + beta[...]

    x2 = jnp.sum(flat * flat, axis=1, keepdims=True)
    m2 = jnp.dot(flat * (-2.0), cbt[...], preferred_element_type=jnp.float32)
    d = (x2 + e2_scr[...]) + m2
    idxs = jnp.argmin(d, axis=1)[:, None]
    idx_out[0, :, :] = idxs
    iota = lax.broadcasted_iota(jnp.int32, (_R1, _K), 1)
    enc_out[...] = (iota == idxs).astype(jnp.float32)


_front = pl.pallas_call(
    _front_body,
    grid=(_G1,),
    in_specs=[
        pl.BlockSpec((_R1, _D), lambda i: (i, 0)),
        pl.BlockSpec((_D, _D), lambda i: (0, 0)),
        pl.BlockSpec((1, _D), lambda i: (0, 0)),
        pl.BlockSpec((_D, _CD), lambda i: (0, 0)),
        pl.BlockSpec((1, _CD), lambda i: (0, 0)),
        pl.BlockSpec((1, _CD), lambda i: (0, 0)),
        pl.BlockSpec((1, _CD), lambda i: (0, 0)),
        pl.BlockSpec((_CD, _K), lambda i: (0, 0)),
    ],
    out_specs=[
        pl.BlockSpec((1, _R1, 1), lambda i: (i, 0, 0)),
        pl.BlockSpec((_R1, _K), lambda i: (i, 0)),
    ],
    out_shape=[
        jax.ShapeDtypeStruct((_G1, _R1, 1), jnp.int32),
        jax.ShapeDtypeStruct((_N, _K), jnp.float32),
    ],
    scratch_shapes=[pltpu.VMEM((1, _K), jnp.float32)],
    compiler_params=pltpu.CompilerParams(dimension_semantics=("arbitrary",)),
)

_NC, _NS = 2, 16
_NW = _NC * _NS
_RPW = _N // _NW
_CH = 128
_NCH = _RPW // _CH


def _gather_body(cb_hbm, idx_hbm, out_hbm, idx_v, rows_v, gsem, ssem):
    wid = lax.axis_index("s") * _NC + lax.axis_index("c")
    base = wid * _RPW
    pltpu.sync_copy(idx_hbm.at[pl.ds(base, _RPW)], idx_v)

    def g(ch, buf):
        return pltpu.async_copy(
            cb_hbm.at[idx_v.at[pl.ds(ch * _CH, _CH)]], rows_v.at[buf], gsem)

    def s(ch, buf):
        return pltpu.async_copy(
            rows_v.at[buf], out_hbm.at[pl.ds(base + ch * _CH, _CH)], ssem)

    g0 = g(0, 0)
    g1 = g(1, 1)
    g0.wait()
    s0 = s(0, 0)
    g2 = g(2, 2)
    g1.wait()
    s1 = s(1, 1)
    s0.wait()
    g3 = g(3, 0)
    g2.wait()
    s2 = s(2, 2)
    g3.wait()
    s3 = s(3, 0)
    s1.wait()
    s2.wait()
    s3.wait()


@functools.cache
def _build_gather():
    return functools.partial(
        pl.kernel,
        out_type=jax.ShapeDtypeStruct((_N, _CD), jnp.float32),
        mesh=plsc.VectorSubcoreMesh(core_axis_name="c", subcore_axis_name="s"),
        scratch_types=[
            pltpu.VMEM((_RPW,), jnp.int32),
            pltpu.VMEM((3, _CH, _CD), jnp.float32),
            pltpu.SemaphoreType.DMA,
            pltpu.SemaphoreType.DMA,
        ],
    )(_gather_body)


def _gather(cb, idx):
    return _build_gather()(cb, idx)

_R3 = 1024
_G3 = _N // _R3


def _back_body(qr, wo1, bo1, wo2, bo2, g, beta, out):
    h = jnp.maximum(jnp.dot(qr[...], wo1[...], preferred_element_type=jnp.float32)
                    + bo1[...], 0.0)
    h = jnp.dot(h, wo2[...], preferred_element_type=jnp.float32) + bo2[...]
    mu = jnp.mean(h, axis=1, keepdims=True)
    var = jnp.mean((h - mu) ** 2, axis=1, keepdims=True)
    out[...] = (h - mu) / jnp.sqrt(var + 1e-5) * g[...] + beta[...]


_back = pl.pallas_call(
    _back_body,
    grid=(_G3,),
    in_specs=[
        pl.BlockSpec((_R3, _CD), lambda i: (i, 0)),
        pl.BlockSpec((_CD, _D), lambda i: (0, 0)),
        pl.BlockSpec((1, _D), lambda i: (0, 0)),
        pl.BlockSpec((_D, _D), lambda i: (0, 0)),
        pl.BlockSpec((1, _D), lambda i: (0, 0)),
        pl.BlockSpec((1, _D), lambda i: (0, 0)),
        pl.BlockSpec((1, _D), lambda i: (0, 0)),
    ],
    out_specs=pl.BlockSpec((_R3, _D), lambda i: (i, 0)),
    out_shape=jax.ShapeDtypeStruct((_N, _D), jnp.float32),
    compiler_params=pltpu.CompilerParams(dimension_semantics=("arbitrary",)),
)


def kernel(features, W_in1, b_in1, W_in2, b_in2, g_nin, beta_nin, codebook,
           W_out1, b_out1, W_out2, b_out2, g_nout, beta_nout):
    feat = features.reshape(_N, _D)
    cbt = codebook.T
    idx3, enc = _front(feat, W_in1, b_in1.reshape(1, -1), W_in2,
                       b_in2.reshape(1, -1), g_nin.reshape(1, -1),
                       beta_nin.reshape(1, -1), cbt)
    idx_flat = idx3.reshape(_N)
    qr = _gather(codebook, idx_flat)
    q = _back(qr, W_out1, b_out1.reshape(1, -1), W_out2,
              b_out2.reshape(1, -1), g_nout.reshape(1, -1),
              beta_nout.reshape(1, -1))
    return q.reshape(_B, _T, _D), idx_flat.reshape(-1, 1), enc


# speedup vs baseline: 1.5805x; 1.0251x over previous
import functools

import jax
import jax.numpy as jnp
from jax import lax
from jax.experimental import pallas as pl
from jax.experimental.pallas import tpu as pltpu
from jax.experimental.pallas import tpu_sc as plsc

_B, _T, _D, _CD, _K = 16, 1024, 768, 256, 8192
_N = _B * _T
_H = _N // 2

_R1 = 512
_GH = _H // _R1


def _front_body(feat, w1, b1, w2, b2, g, beta, cbt, idx_out, enc_out, e2_scr):
    @pl.when(pl.program_id(0) == 0)
    def _():
        c = cbt[...]
        e2_scr[...] = jnp.sum(c * c, axis=0, keepdims=True)

    x = feat[...]
    h = jnp.maximum(jnp.dot(x, w1[...], preferred_element_type=jnp.float32)
                    + b1[...], 0.0)
    h = jnp.dot(h, w2[...], preferred_element_type=jnp.float32) + b2[...]
    mu = jnp.mean(h, axis=1, keepdims=True)
    var = jnp.mean((h - mu) ** 2, axis=1, keepdims=True)
    flat = (h - mu) / jnp.sqrt(var + 1e-5) * g[...] + beta[...]

    x2 = jnp.sum(flat * flat, axis=1, keepdims=True)
    m2 = jnp.dot(flat * (-2.0), cbt[...], preferred_element_type=jnp.float32)
    d = (x2 + e2_scr[...]) + m2
    idxs = jnp.argmin(d, axis=1)[:, None]
    idx_out[0, :, :] = idxs
    iota = lax.broadcasted_iota(jnp.int32, (_R1, _K), 1)
    enc_out[...] = (iota == idxs).astype(jnp.float32)


def _front_body_b(feat, w1, b1, w2, b2, g, beta, cbt, enc_prev, idx_out,
                  enc_out, e2_scr):
    _front_body(feat, w1, b1, w2, b2, g, beta, cbt, idx_out, enc_out, e2_scr)


def _front_call(half):
    off = half * _GH
    body = _front_body if half == 0 else _front_body_b
    in_specs = [
        pl.BlockSpec((_R1, _D), lambda i: (i + off, 0)),
        pl.BlockSpec((_D, _D), lambda i: (0, 0)),
        pl.BlockSpec((1, _D), lambda i: (0, 0)),
        pl.BlockSpec((_D, _CD), lambda i: (0, 0)),
        pl.BlockSpec((1, _CD), lambda i: (0, 0)),
        pl.BlockSpec((1, _CD), lambda i: (0, 0)),
        pl.BlockSpec((1, _CD), lambda i: (0, 0)),
        pl.BlockSpec((_CD, _K), lambda i: (0, 0)),
    ]
    kwargs = {}
    if half == 1:
        in_specs.append(pl.BlockSpec(memory_space=pl.ANY))
        kwargs["input_output_aliases"] = {8: 1}
    return pl.pallas_call(
        body,
        grid=(_GH,),
        in_specs=in_specs,
        out_specs=[
            pl.BlockSpec((1, _R1, 1), lambda i: (i, 0, 0)),
            pl.BlockSpec((_R1, _K), lambda i: (i + off, 0)),
        ],
        out_shape=[
            jax.ShapeDtypeStruct((_GH, _R1, 1), jnp.int32),
            jax.ShapeDtypeStruct((_N, _K), jnp.float32),
        ],
        scratch_shapes=[pltpu.VMEM((1, _K), jnp.float32)],
        compiler_params=pltpu.CompilerParams(
            dimension_semantics=("arbitrary",)),
        **kwargs,
    )


_frontA = _front_call(0)
_frontB = _front_call(1)

_NC, _NS = 2, 16
_NW = _NC * _NS
_RPW = _H // _NW
_CH = 128
_NCH = _RPW // _CH


def _gather_body(cb_hbm, idx_hbm, out_hbm, idx_v, rows_v, gsem, ssem):
    wid = lax.axis_index("s") * _NC + lax.axis_index("c")
    base = wid * _RPW
    pltpu.sync_copy(idx_hbm.at[pl.ds(base, _RPW)], idx_v)

    def g(ch, buf):
        return pltpu.async_copy(
            cb_hbm.at[idx_v.at[pl.ds(ch * _CH, _CH)]], rows_v.at[buf], gsem)

    def s(ch, buf):
        return pltpu.async_copy(
            rows_v.at[buf], out_hbm.at[pl.ds(base + ch * _CH, _CH)], ssem)

    g0 = g(0, 0)
    g1 = g(1, 1)
    g0.wait()
    s0 = s(0, 0)
    g1.wait()
    s1 = s(1, 1)
    s0.wait()
    s1.wait()


@functools.cache
def _build_gather():
    return functools.partial(
        pl.kernel,
        out_type=jax.ShapeDtypeStruct((_H, _CD), jnp.float32),
        mesh=plsc.VectorSubcoreMesh(core_axis_name="c", subcore_axis_name="s"),
        scratch_types=[
            pltpu.VMEM((_RPW,), jnp.int32),
            pltpu.VMEM((2, _CH, _CD), jnp.float32),
            pltpu.SemaphoreType.DMA,
            pltpu.SemaphoreType.DMA,
        ],
    )(_gather_body)


def _gather(cb, idx):
    return _build_gather()(cb, idx)

_R3 = 1024
_G3 = _H // _R3


def _back_body(qr, wo1, bo1, wo2, bo2, g, beta, out):
    h = jnp.maximum(jnp.dot(qr[...], wo1[...], preferred_element_type=jnp.float32)
                    + bo1[...], 0.0)
    h = jnp.dot(h, wo2[...], preferred_element_type=jnp.float32) + bo2[...]
    mu = jnp.mean(h, axis=1, keepdims=True)
    var = jnp.mean((h - mu) ** 2, axis=1, keepdims=True)
    out[...] = (h - mu) / jnp.sqrt(var + 1e-5) * g[...] + beta[...]


def _back_body_b(qr, wo1, bo1, wo2, bo2, g, beta, q_prev, out):
    _back_body(qr, wo1, bo1, wo2, bo2, g, beta, out)


def _back_call(half):
    off = half * _G3
    body = _back_body if half == 0 else _back_body_b
    in_specs = [
        pl.BlockSpec((_R3, _CD), lambda i: (i, 0)),
        pl.BlockSpec((_CD, _D), lambda i: (0, 0)),
        pl.BlockSpec((1, _D), lambda i: (0, 0)),
        pl.BlockSpec((_D, _D), lambda i: (0, 0)),
        pl.BlockSpec((1, _D), lambda i: (0, 0)),
        pl.BlockSpec((1, _D), lambda i: (0, 0)),
        pl.BlockSpec((1, _D), lambda i: (0, 0)),
    ]
    kwargs = {}
    if half == 1:
        in_specs.append(pl.BlockSpec(memory_space=pl.ANY))
        kwargs["input_output_aliases"] = {7: 0}
    return pl.pallas_call(
        body,
        grid=(_G3,),
        in_specs=in_specs,
        out_specs=pl.BlockSpec((_R3, _D), lambda i: (i + off, 0)),
        out_shape=jax.ShapeDtypeStruct((_N, _D), jnp.float32),
        compiler_params=pltpu.CompilerParams(
            dimension_semantics=("arbitrary",)),
        **kwargs,
    )


_backA = _back_call(0)
_backB = _back_call(1)


def kernel(features, W_in1, b_in1, W_in2, b_in2, g_nin, beta_nin, codebook,
           W_out1, b_out1, W_out2, b_out2, g_nout, beta_nout):
    feat = features.reshape(_N, _D)
    cbt = codebook.T
    wargs = (W_in1, b_in1.reshape(1, -1), W_in2, b_in2.reshape(1, -1),
             g_nin.reshape(1, -1), beta_nin.reshape(1, -1), cbt)
    idxA3, enc_part = _frontA(feat, *wargs)
    idxB3, enc = _frontB(feat, *wargs, enc_part)
    idxA = idxA3.reshape(_H)
    idxB = idxB3.reshape(_H)
    qrA = _gather(codebook, idxA)
    qrB = _gather(codebook, idxB)
    oargs = (W_out1, b_out1.reshape(1, -1), W_out2, b_out2.reshape(1, -1),
             g_nout.reshape(1, -1), beta_nout.reshape(1, -1))
    qA = _backA(qrA, *oargs)
    q = _backB(qrB, *oargs, qA)
    idx_flat = jnp.concatenate([idxA, idxB])
    return q.reshape(_B, _T, _D), idx_flat.reshape(-1, 1), enc
